# baseline (speedup 1.0000x reference)
import functools

import jax
import jax.numpy as jnp
from jax import lax
from jax.experimental import pallas as pl
from jax.experimental.pallas import tpu as pltpu
from jax.experimental.pallas import tpu_sc as plsc

N = 10000
NP = 10240
D = 128
E = 320000
EPS = 1e-5

NC = 2
NS = 16
NW = NC * NS
NH = NP // NC
NACC = NH + 128
ERS = 2560
EP = ERS * 128
RPW = ERS // NS
GRP = 8
NGRP = RPW // GRP
CAP = 168
CAPX = CAP + 8
RPT = NACC // NS

_GDN = lax.GatherDimensionNumbers(
    offset_dims=(), collapsed_slice_dims=(0,), start_index_map=(0,))


def _take16(x, idx):
    return lax.gather(x, idx[:, None], _GDN, (1,),
                      mode=lax.GatherScatterMode.PROMISE_IN_BOUNDS)


def _scan16(x, lane):
    for dd in (1, 2, 4, 8):
        shifted = _take16(x, jnp.maximum(lane - dd, 0))
        x = x + jnp.where(lane >= dd, shifted, 0)
    return x


_mesh = plsc.VectorSubcoreMesh(
    core_axis_name="c", subcore_axis_name="s", num_cores=NC, num_subcores=NS)


@functools.partial(
    pl.kernel,
    out_type=[
        jax.ShapeDtypeStruct((NC * NH // 128, 128), jnp.float32),
        jax.ShapeDtypeStruct((NW, CAPX, 128), jnp.int32),
        jax.ShapeDtypeStruct((NW, CAPX, 128), jnp.int32),
    ],
    mesh=_mesh,
    scratch_types=[
        pltpu.VMEM((128,), jnp.float32),
        pltpu.VMEM((GRP, 128), jnp.int32),
        pltpu.VMEM((GRP, 128), jnp.int32),
        pltpu.VMEM((GRP, 128), jnp.int32),
        pltpu.VMEM((GRP, 128), jnp.int32),
        pltpu.VMEM((1024,), jnp.int32),
        pltpu.VMEM((1024,), jnp.int32),
        pltpu.VMEM((CAPX * 128,), jnp.int32),
        pltpu.VMEM((CAPX, 128), jnp.int32),
        pltpu.VMEM((NH,), jnp.float32),
        pltpu.VMEM((NH // 128, 128), jnp.float32),
        pltpu.VMEM((RPT,), jnp.float32),
        pltpu.VMEM_SHARED((NACC,), jnp.float32),
        pltpu.VMEM_SHARED((NS * CAPX * 128,), jnp.int32),
        pltpu.VMEM_SHARED((NS * CAPX * 128,), jnp.int32),
        pltpu.SemaphoreType.DMA,
    ],
)
def _prep_kernel(src_hbm, dst_hbm, deg_hbm, csrc_hbm, cdst_hbm,
                 ones_v, sidx, didx, dloc, posb, tv0, tvn, lin, sq,
                 dlin, dsq, zbuf, dacc, sstg, dstg, sem):
    c = lax.axis_index("c")
    s = lax.axis_index("s")
    w = c * NS + s
    lo = c * NH
    sbase = s * CAPX * 128

    def _zero(i, carry):
        zbuf[pl.ds(i * 16, 16)] = jnp.zeros((16,), jnp.float32)
        return carry
    lax.fori_loop(0, RPT // 16, _zero, 0)
    for i in range(8):
        ones_v[pl.ds(i * 16, 16)] = jnp.ones((16,), jnp.float32)

    def _fill(i, carry):
        tv0[pl.ds(i * 16, 16)] = jnp.zeros((16,), jnp.int32)
        tvn[pl.ds(i * 16, 16)] = jnp.full((16,), NH, jnp.int32)
        return carry
    lax.fori_loop(0, 64, _fill, 0)

    pltpu.sync_copy(zbuf, dacc.at[pl.ds(s * RPT, RPT)])
    for i in range(CAPX // 8):
        pltpu.sync_copy(tv0, sstg.at[pl.ds(sbase + i * 1024, 1024)])
        pltpu.sync_copy(tvn, dstg.at[pl.ds(sbase + i * 1024, 1024)])
    plsc.subcore_barrier()

    r0 = s * RPW
    lane = lax.iota(jnp.int32, 16)

    def _grp(g, kvec):
        base = r0 + g * GRP
        pltpu.sync_copy(src_hbm.at[pl.ds(base, GRP)], sidx)
        pltpu.sync_copy(dst_hbm.at[pl.ds(base, GRP)], didx)
        cps = []
        for j in range(GRP):
            for q in range(8):
                dv = didx[j, pl.ds(q * 16, 16)] - lo
                keep = (dv >= 0) & (dv < NH)
                dloc[j, pl.ds(q * 16, 16)] = jnp.where(keep, dv, NH)
                ks = _scan16(jnp.where(keep, dv - dv + 1, dv - dv), lane)
                posb[j, pl.ds(q * 16, 16)] = sbase + jnp.where(
                    keep, kvec + ks - 1, CAPX * 128 - 1)
                kvec = kvec + _take16(ks, lane - lane + 15)
            cps.append(pltpu.async_copy(
                sidx.at[j], sstg.at[posb.at[j]], sem))
            cps.append(pltpu.async_copy(
                dloc.at[j], dstg.at[posb.at[j]], sem))
            cps.append(pltpu.async_copy(
                ones_v, dacc.at[dloc.at[j]], sem, add=True))
        for cp in cps:
            cp.wait()
        return kvec
    kvec = lax.fori_loop(0, NGRP, _grp, jnp.zeros((16,), jnp.int32))

    ngv = (kvec + 1023) >> 10
    for i in range(8):
        tv0[pl.ds(i * 16, 16)] = ngv
    pltpu.sync_copy(tv0.at[pl.ds(0, 128)],
                    dstg.at[pl.ds(sbase + CAP * 128, 128)])

    for src_sp, out in ((sstg, csrc_hbm), (dstg, cdst_hbm)):
        pltpu.sync_copy(src_sp.at[pl.ds(sbase, CAPX * 128)], lin)
        def _relay(t, carry):
            sq[t // 8, pl.ds((t % 8) * 16, 16)] = (
                lin[pl.ds(t * 16, 16)])
            return carry
        lax.fori_loop(0, CAPX * 8, _relay, 0)
        pltpu.sync_copy(sq, out.at[w])

    plsc.subcore_barrier()

    @pl.when(s == 0)
    def _():
        pltpu.sync_copy(dacc.at[pl.ds(0, NH)], dlin)
        for r in range(NH // 128):
            for q in range(8):
                dsq[r, pl.ds(q * 16, 16)] = dlin[pl.ds(r * 128 + q * 16, 16)]
        pltpu.sync_copy(dsq, deg_hbm.at[pl.ds(c * (NH // 128), NH // 128)])


@functools.partial(
    pl.kernel,
    out_type=jax.ShapeDtypeStruct((NC, NH, D), jnp.float32),
    mesh=_mesh,
    scratch_types=[
        pltpu.VMEM((256, D), jnp.float32),
        pltpu.VMEM((256, D), jnp.float32),
        pltpu.VMEM((GRP, 128), jnp.int32),
        pltpu.VMEM((GRP, 128), jnp.int32),
        pltpu.VMEM((8, 128), jnp.int32),
        pltpu.VMEM_SHARED((NACC, D), jnp.float32),
        pltpu.SemaphoreType.DMA,
        pltpu.SemaphoreType.DMA,
    ],
)
def _spmm_kernel(csrc_hbm, cdst_hbm, table_hbm, out_hbm,
                 bufa, bufb, sidx, dloc, ngrow, acc, gsem, ssem):
    c = lax.axis_index("c")
    s = lax.axis_index("s")
    w = c * NS + s

    def _zero(t, carry):
        bufa[t // 8, pl.ds((t % 8) * 16, 16)] = jnp.zeros((16,), jnp.float32)
        return carry
    lax.fori_loop(0, 256 * 8, _zero, 0)
    pltpu.sync_copy(bufa, acc.at[pl.ds(s * RPT, 256)])
    pltpu.sync_copy(bufa.at[pl.ds(0, RPT - 256)],
                    acc.at[pl.ds(s * RPT + 256, RPT - 256)])
    plsc.subcore_barrier()

    pltpu.sync_copy(cdst_hbm.at[w, pl.ds(CAP, 8)], ngrow)
    ng = ngrow[0, pl.ds(0, 16)][0]

    def _grp(g, carry):
        pltpu.sync_copy(csrc_hbm.at[w, pl.ds(g * GRP, GRP)], sidx)
        pltpu.sync_copy(cdst_hbm.at[w, pl.ds(g * GRP, GRP)], dloc)
        scat = {}
        for h in range(4):
            buf = bufa if h % 2 == 0 else bufb
            if h >= 2:
                for cp in scat[h - 2]:
                    cp.wait()
            gs = [pltpu.async_copy(table_hbm.at[sidx.at[2 * h + j]],
                                   buf.at[pl.ds(j * 128, 128)], gsem)
                  for j in range(2)]
            for cp in gs:
                cp.wait()
            scat[h] = [pltpu.async_copy(buf.at[pl.ds(j * 128, 128)],
                                        acc.at[dloc.at[2 * h + j]], ssem,
                                        add=True)
                       for j in range(2)]
        for cp in scat[2] + scat[3]:
            cp.wait()
        return carry
    lax.fori_loop(0, ng, _grp, 0)

    plsc.subcore_barrier()
    pltpu.sync_copy(acc.at[pl.ds(s * (NH // NS), NH // NS)],
                    out_hbm.at[c, pl.ds(s * (NH // NS), NH // NS)])


_BLK = 1024
_GRID = NP // _BLK

def _dot(a, b):
    return lax.dot_general(a, b, (((1,), (0,)), ((), ())),
                           precision=lax.Precision.HIGHEST,
                           preferred_element_type=jnp.float32)


def _k0_body(x_ref, d_ref, w_ref, o_ref):
    o_ref[...] = _dot(d_ref[...] * x_ref[...], w_ref[...])


def _kmid_body(acc_ref, g_ref, d_ref, w_ref, t_ref, c1_ref, o_ref):
    u = d_ref[...] * (acc_ref[...] + g_ref[...])
    h = jnp.maximum(t_ref[...] * u + c1_ref[...], 0.0)
    o_ref[...] = _dot(d_ref[...] * h, w_ref[...])


def _k3_body(acc_ref, g_ref, d_ref, b_ref, o_ref):
    o_ref[...] = d_ref[...] * (acc_ref[...] + g_ref[...]) + b_ref[...]


_row_spec = pl.BlockSpec((_BLK, D), lambda i: (i, 0))
_mat_spec = pl.BlockSpec((D, D), lambda i: (0, 0))
_vec_spec = pl.BlockSpec((1, D), lambda i: (0, 0))
_out_sds = jax.ShapeDtypeStruct((NP, D), jnp.float32)

_k0 = pl.pallas_call(
    _k0_body, grid=(_GRID,),
    in_specs=[_row_spec, _row_spec, _mat_spec],
    out_specs=_row_spec, out_shape=_out_sds)

_kmid = pl.pallas_call(
    _kmid_body, grid=(_GRID,),
    in_specs=[_row_spec, _row_spec, _row_spec, _mat_spec, _vec_spec,
              _vec_spec],
    out_specs=_row_spec, out_shape=_out_sds)

_k3 = pl.pallas_call(
    _k3_body, grid=(_GRID,),
    in_specs=[_row_spec, _row_spec, _row_spec, _vec_spec],
    out_specs=_row_spec, out_shape=_out_sds)


def kernel(x, edge_index, W0, b0, g0, be0, W1, b1, g1, be1, W2, b2):
    src = edge_index[0]
    dst = edge_index[1]
    pad = EP - E
    srcp = jnp.concatenate(
        [src, jnp.zeros((pad,), src.dtype)]).reshape(ERS, 128)
    dstp = jnp.concatenate(
        [dst, jnp.full((pad,), N, dst.dtype)]).reshape(ERS, 128)
    xp = jnp.concatenate([x, jnp.zeros((NP - N, D), x.dtype)], axis=0)

    deg2d, csrc, cdst = _prep_kernel(srcp, dstp)
    dinv = lax.rsqrt(deg2d.reshape(NP) + 1.0)
    dinv2d = jnp.broadcast_to(dinv[:, None], (NP, D))

    cbn = (1.0 + EPS) ** -0.5
    t0 = (g0 * cbn).reshape(1, D)
    c10 = (t0[0] * b0 + be0).reshape(1, D)
    t1 = (g1 * cbn).reshape(1, D)
    c11 = (t1[0] * b1 + be1).reshape(1, D)
    b2r = b2.reshape(1, D)

    G0 = _k0(xp, dinv2d, W0)
    acc = _spmm_kernel(csrc, cdst, G0).reshape(NP, D)
    G1 = _kmid(acc, G0, dinv2d, W1, t0, c10)
    acc = _spmm_kernel(csrc, cdst, G1).reshape(NP, D)
    G2 = _kmid(acc, G1, dinv2d, W2, t1, c11)
    acc = _spmm_kernel(csrc, cdst, G2).reshape(NP, D)
    z = _k3(acc, G2, dinv2d, b2r)
    return z[:N]


# speedup vs baseline: 1.6069x; 1.6069x over previous
import functools

import jax
import jax.numpy as jnp
from jax import lax
from jax.experimental import pallas as pl
from jax.experimental.pallas import tpu as pltpu
from jax.experimental.pallas import tpu_sc as plsc

N = 10000
NP = 10240
D = 128
E = 320000
EPS = 1e-5

NC = 2
NS = 16
NW = NC * NS
NH = NP // NC
NACC = NH + 128
ERS = 2560
EP = ERS * 128
RPW = ERS // NS
GRP = 8
NGRP = RPW // GRP
CAP = 168
CAPX = CAP + 8
RPT = NACC // NS

_GDN = lax.GatherDimensionNumbers(
    offset_dims=(), collapsed_slice_dims=(0,), start_index_map=(0,))


def _take16(x, idx):
    return lax.gather(x, idx[:, None], _GDN, (1,),
                      mode=lax.GatherScatterMode.PROMISE_IN_BOUNDS)


def _scan16(x, lane):
    for dd in (1, 2, 4, 8):
        shifted = _take16(x, jnp.maximum(lane - dd, 0))
        x = x + jnp.where(lane >= dd, shifted, 0)
    return x


_mesh = plsc.VectorSubcoreMesh(
    core_axis_name="c", subcore_axis_name="s", num_cores=NC, num_subcores=NS)


@functools.partial(
    pl.kernel,
    out_type=[
        jax.ShapeDtypeStruct((NC * NH // 128, 128), jnp.float32),
        jax.ShapeDtypeStruct((NW, CAPX, 128), jnp.int32),
    ],
    mesh=_mesh,
    scratch_types=[
        pltpu.VMEM((128,), jnp.float32),
        pltpu.VMEM((GRP, 128), jnp.int32),
        pltpu.VMEM((GRP, 128), jnp.int32),
        pltpu.VMEM((GRP, 128), jnp.int32),
        pltpu.VMEM((GRP, 128), jnp.int32),
        pltpu.VMEM((GRP, 128), jnp.int32),
        pltpu.VMEM((1024,), jnp.int32),
        pltpu.VMEM((CAPX * 128,), jnp.int32),
        pltpu.VMEM((CAPX, 128), jnp.int32),
        pltpu.VMEM((NH,), jnp.float32),
        pltpu.VMEM((NH // 128, 128), jnp.float32),
        pltpu.VMEM((RPT,), jnp.float32),
        pltpu.VMEM_SHARED((NACC,), jnp.float32),
        pltpu.VMEM_SHARED((NS * CAPX * 128,), jnp.int32),
        pltpu.SemaphoreType.DMA,
    ],
)
def _prep_kernel(src_hbm, dst_hbm, deg_hbm, cpk_hbm,
                 ones_v, sidx, didx, dloc, pk, posb, tvn, lin, sq,
                 dlin, dsq, zbuf, dacc, pstg, sem):
    c = lax.axis_index("c")
    s = lax.axis_index("s")
    w = c * NS + s
    lo = c * NH
    sbase = s * CAPX * 128

    def _zero(i, carry):
        zbuf[pl.ds(i * 16, 16)] = jnp.zeros((16,), jnp.float32)
        return carry
    lax.fori_loop(0, RPT // 16, _zero, 0)
    for i in range(8):
        ones_v[pl.ds(i * 16, 16)] = jnp.ones((16,), jnp.float32)

    lane = lax.iota(jnp.int32, 16)

    def _fill(i, carry):
        tvn[pl.ds(i * 16, 16)] = NH + ((i * 16 + lane) & 127)
        return carry
    lax.fori_loop(0, 64, _fill, 0)

    pltpu.sync_copy(zbuf, dacc.at[pl.ds(s * RPT, RPT)])
    for i in range(CAPX // 8):
        pltpu.sync_copy(tvn, pstg.at[pl.ds(sbase + i * 1024, 1024)])
    plsc.subcore_barrier()

    r0 = s * RPW

    def _grp(g, kvec):
        base = r0 + g * GRP
        pltpu.sync_copy(src_hbm.at[pl.ds(base, GRP)], sidx)
        pltpu.sync_copy(dst_hbm.at[pl.ds(base, GRP)], didx)
        cps = []
        for j in range(GRP):
            for q in range(8):
                dv = didx[j, pl.ds(q * 16, 16)] - lo
                sv = sidx[j, pl.ds(q * 16, 16)]
                keep = (dv >= 0) & (dv < NH)
                dl = jnp.where(keep, dv, NH + ((q * 16 + lane) & 127))
                dloc[j, pl.ds(q * 16, 16)] = dl
                pk[j, pl.ds(q * 16, 16)] = (sv << 13) | dl
                ks = _scan16(jnp.where(keep, dv - dv + 1, dv - dv), lane)
                posb[j, pl.ds(q * 16, 16)] = sbase + jnp.where(
                    keep, kvec + ks - 1, (CAP + 1) * 128 + q * 16 + lane)
                kvec = kvec + _take16(ks, lane - lane + 15)
            cps.append(pltpu.async_copy(
                pk.at[j], pstg.at[posb.at[j]], sem))
            cps.append(pltpu.async_copy(
                ones_v, dacc.at[dloc.at[j]], sem, add=True))
        for cp in cps:
            cp.wait()
        return kvec
    kvec = lax.fori_loop(0, NGRP, _grp, jnp.zeros((16,), jnp.int32))

    ngv = (kvec + 1023) >> 10
    for i in range(8):
        tvn[pl.ds(i * 16, 16)] = ngv
    pltpu.sync_copy(tvn.at[pl.ds(0, 128)],
                    pstg.at[pl.ds(sbase + CAP * 128, 128)])

    pltpu.sync_copy(pstg.at[pl.ds(sbase, CAPX * 128)], lin)

    def _relay(t, carry):
        sq[t // 8, pl.ds((t % 8) * 16, 16)] = lin[pl.ds(t * 16, 16)]
        return carry
    lax.fori_loop(0, CAPX * 8, _relay, 0)
    pltpu.sync_copy(sq, cpk_hbm.at[w])

    plsc.subcore_barrier()

    @pl.when(s == 0)
    def _():
        pltpu.sync_copy(dacc.at[pl.ds(0, NH)], dlin)
        for r in range(NH // 128):
            for q in range(8):
                dsq[r, pl.ds(q * 16, 16)] = dlin[pl.ds(r * 128 + q * 16, 16)]
        pltpu.sync_copy(dsq, deg_hbm.at[pl.ds(c * (NH // 128), NH // 128)])


@functools.partial(
    pl.kernel,
    out_type=jax.ShapeDtypeStruct((NC, NH, D), jnp.float32),
    mesh=_mesh,
    scratch_types=[
        pltpu.VMEM((128, D), jnp.float32),
        pltpu.VMEM((128, D), jnp.float32),
        pltpu.VMEM((128, D), jnp.float32),
        pltpu.VMEM((128, D), jnp.float32),
        pltpu.VMEM((128, D), jnp.float32),
        pltpu.VMEM((GRP, 128), jnp.int32),
        pltpu.VMEM((GRP, 128), jnp.int32),
        pltpu.VMEM((GRP, 128), jnp.int32),
        pltpu.VMEM_SHARED((NACC, D), jnp.float32),
        pltpu.SemaphoreType.DMA,
        pltpu.SemaphoreType.DMA,
        pltpu.SemaphoreType.DMA,
        pltpu.SemaphoreType.DMA,
        pltpu.SemaphoreType.DMA,
        pltpu.SemaphoreType.DMA,
        pltpu.SemaphoreType.DMA,
        pltpu.SemaphoreType.DMA,
        pltpu.SemaphoreType.DMA,
        pltpu.SemaphoreType.DMA,
    ],
)
def _spmm_kernel(cpk_hbm, table_hbm, out_hbm,
                 b0, b1, b2, b3, b4, pk, sidx, dloc, acc,
                 g0, g1, g2, g3, g4, s0, s1, s2, s3, s4):
    slots = (b0, b1, b2, b3, b4)
    gsems = (g0, g1, g2, g3, g4)
    ssems = (s0, s1, s2, s3, s4)
    c = lax.axis_index("c")
    s = lax.axis_index("s")
    w = c * NS + s

    def _zero(t, carry):
        slots[0][t // 8, pl.ds((t % 8) * 16, 16)] = jnp.zeros(
            (16,), jnp.float32)
        return carry
    lax.fori_loop(0, 128 * 8, _zero, 0)
    pltpu.sync_copy(slots[0], acc.at[pl.ds(s * RPT, 128)])
    pltpu.sync_copy(slots[0], acc.at[pl.ds(s * RPT + 128, 128)])
    pltpu.sync_copy(slots[0].at[pl.ds(0, RPT - 256)],
                    acc.at[pl.ds(s * RPT + 256, RPT - 256)])
    plsc.subcore_barrier()

    pltpu.sync_copy(cpk_hbm.at[w, pl.ds(CAP, 8)], pk)
    ng = pk[0, pl.ds(0, 16)][0]

    def _gather(r):
        return pltpu.async_copy(table_hbm.at[sidx.at[r]], slots[r % 5],
                                gsems[r % 5])

    def _scatter(r):
        return pltpu.async_copy(slots[r % 5], acc.at[dloc.at[r]],
                                ssems[r % 5], add=True)

    def _grp(g, carry):
        pltpu.sync_copy(cpk_hbm.at[w, pl.ds(g * GRP, GRP)], pk)
        for j in range(GRP):
            for q in range(8):
                pv = pk[j, pl.ds(q * 16, 16)]
                sidx[j, pl.ds(q * 16, 16)] = pv >> 13
                dloc[j, pl.ds(q * 16, 16)] = pv & 8191
        gth = {r: _gather(r) for r in range(3)}
        scat = {}
        for r in range(GRP):
            gth[r].wait()
            scat[r] = _scatter(r)
            nxt = r + 3
            if nxt < GRP:
                if nxt >= 5:
                    scat[nxt - 5].wait()
                gth[nxt] = _gather(nxt)
        for r in range(3, GRP):
            scat[r].wait()
        return carry
    lax.fori_loop(0, ng, _grp, 0)

    plsc.subcore_barrier()
    pltpu.sync_copy(acc.at[pl.ds(s * (NH // NS), NH // NS)],
                    out_hbm.at[c, pl.ds(s * (NH // NS), NH // NS)])


_BLK = 1024
_GRID = NP // _BLK

def _dot(a, b):
    return lax.dot_general(a, b, (((1,), (0,)), ((), ())),
                           precision=lax.Precision.HIGHEST,
                           preferred_element_type=jnp.float32)


def _k0_body(x_ref, d_ref, w_ref, o_ref):
    o_ref[...] = _dot(d_ref[...] * x_ref[...], w_ref[...])


def _kmid_body(acc_ref, g_ref, d_ref, w_ref, t_ref, c1_ref, o_ref):
    u = d_ref[...] * (acc_ref[...] + g_ref[...])
    h = jnp.maximum(t_ref[...] * u + c1_ref[...], 0.0)
    o_ref[...] = _dot(d_ref[...] * h, w_ref[...])


def _k3_body(acc_ref, g_ref, d_ref, b_ref, o_ref):
    o_ref[...] = d_ref[...] * (acc_ref[...] + g_ref[...]) + b_ref[...]


_row_spec = pl.BlockSpec((_BLK, D), lambda i: (i, 0))
_mat_spec = pl.BlockSpec((D, D), lambda i: (0, 0))
_vec_spec = pl.BlockSpec((1, D), lambda i: (0, 0))
_out_sds = jax.ShapeDtypeStruct((NP, D), jnp.float32)

_k0 = pl.pallas_call(
    _k0_body, grid=(_GRID,),
    in_specs=[_row_spec, _row_spec, _mat_spec],
    out_specs=_row_spec, out_shape=_out_sds)

_kmid = pl.pallas_call(
    _kmid_body, grid=(_GRID,),
    in_specs=[_row_spec, _row_spec, _row_spec, _mat_spec, _vec_spec,
              _vec_spec],
    out_specs=_row_spec, out_shape=_out_sds)

_k3 = pl.pallas_call(
    _k3_body, grid=(_GRID,),
    in_specs=[_row_spec, _row_spec, _row_spec, _vec_spec],
    out_specs=_row_spec, out_shape=_out_sds)


def kernel(x, edge_index, W0, b0, g0, be0, W1, b1, g1, be1, W2, b2):
    src = edge_index[0]
    dst = edge_index[1]
    pad = EP - E
    srcp = jnp.concatenate(
        [src, jnp.zeros((pad,), src.dtype)]).reshape(ERS, 128)
    dstp = jnp.concatenate(
        [dst, jnp.full((pad,), NP, dst.dtype)]).reshape(ERS, 128)
    xp = jnp.concatenate([x, jnp.zeros((NP - N, D), x.dtype)], axis=0)

    deg2d, cpk = _prep_kernel(srcp, dstp)
    dinv = lax.rsqrt(deg2d.reshape(NP) + 1.0)
    dinv2d = jnp.broadcast_to(dinv[:, None], (NP, D))

    cbn = (1.0 + EPS) ** -0.5
    t0 = (g0 * cbn).reshape(1, D)
    c10 = (t0[0] * b0 + be0).reshape(1, D)
    t1 = (g1 * cbn).reshape(1, D)
    c11 = (t1[0] * b1 + be1).reshape(1, D)
    b2r = b2.reshape(1, D)

    G0 = _k0(xp, dinv2d, W0)
    acc = _spmm_kernel(cpk, G0).reshape(NP, D)
    G1 = _kmid(acc, G0, dinv2d, W1, t0, c10)
    acc = _spmm_kernel(cpk, G1).reshape(NP, D)
    G2 = _kmid(acc, G1, dinv2d, W2, t1, c11)
    acc = _spmm_kernel(cpk, G2).reshape(NP, D)
    z = _k3(acc, G2, dinv2d, b2r)
    return z[:N]
